# Initial kernel scaffold; baseline (speedup 1.0000x reference)
import jax
import jax.numpy as jnp
from jax.experimental import pallas as pl


def kernel(user_emb, entity_ids, neigh_ent_ids, neigh_rel_ids, entity_table, relation_table, W):
    raise NotImplementedError("write your pallas kernel here")


# speedup vs baseline: 10.0376x; 10.0376x over previous
import functools

import jax
import jax.numpy as jnp
from jax import lax
from jax.experimental import pallas as pl
from jax.experimental.pallas import tpu as pltpu
from jax.experimental.pallas import tpu_sc as plsc


def _scores_body(u_ref, wt_ref, relt_ref, out_ref):
    proj = jnp.dot(u_ref[...], wt_ref[...], preferred_element_type=jnp.float32)
    out_ref[...] = jnp.dot(proj, relt_ref[...], preferred_element_type=jnp.float32)


def _all_scores(user_emb, W, relation_table):
    B, D = user_emb.shape
    NR = relation_table.shape[0]
    blk = 2048
    grid = (B // blk,)
    return pl.pallas_call(
        _scores_body,
        grid=grid,
        in_specs=[
            pl.BlockSpec((blk, D), lambda i: (i, 0)),
            pl.BlockSpec((D, D), lambda i: (0, 0)),
            pl.BlockSpec((D, NR), lambda i: (0, 0)),
        ],
        out_specs=pl.BlockSpec((blk, NR), lambda i: (i, 0)),
        out_shape=jax.ShapeDtypeStruct((B, NR), jnp.float32),
    )(user_emb, W.T, relation_table.T)


_L = 16
_CH = 8


def _sc_agg(all_scores, entity_ids, neigh_ent_ids, neigh_rel_ids, entity_table):
    B, NR = all_scores.shape
    NS = neigh_ent_ids.shape[1]
    D = entity_table.shape[1]
    try:
        info = plsc.get_sparse_core_info()
        NC, NSC = info.num_cores, info.num_subcores
    except Exception:
        NC, NSC = 2, 16
    NW = NC * NSC
    RW = B // NW
    n_chunks = RW // _CH
    DV = D // _L

    mesh = plsc.VectorSubcoreMesh(
        core_axis_name="c", subcore_axis_name="s",
        num_cores=NC, num_subcores=NSC)

    @functools.partial(
        pl.kernel,
        out_type=jax.ShapeDtypeStruct((B, D), jnp.float32),
        mesh=mesh,
        scratch_types=[
            pltpu.VMEM((RW,), jnp.int32),
            pltpu.VMEM((_CH, NS), jnp.int32),
            pltpu.VMEM((_CH, NS), jnp.int32),
            pltpu.VMEM((_CH * NR,), jnp.float32),
            pltpu.VMEM((_CH * NS,), jnp.float32),
            pltpu.VMEM((_CH, NS, D), jnp.float32),
            pltpu.VMEM((_CH, D), jnp.float32),
            pltpu.VMEM((_CH, D), jnp.float32),
            pltpu.SemaphoreType.DMA,
            pltpu.SemaphoreType.DMA,
        ],
        compiler_params=pltpu.CompilerParams(needs_layout_passes=False),
    )
    def k(sc_hbm, eid_hbm, nid_hbm, rid_hbm, table_hbm, out_hbm,
          eid_v, nid_v, rid_v, sc_v, wbuf, rows_v, self_v, out_v, gsem, csem):
        wid = lax.axis_index("s") * NC + lax.axis_index("c")
        wbase = wid * RW
        pltpu.sync_copy(eid_hbm.at[pl.ds(wbase, RW)], eid_v)

        def chunk_body(c, _):
            base = wbase + c * _CH
            pltpu.sync_copy(nid_hbm.at[pl.ds(base, _CH)], nid_v)
            pltpu.sync_copy(rid_hbm.at[pl.ds(base, _CH)], rid_v)
            pltpu.sync_copy(sc_hbm.at[pl.ds(base * NR, _CH * NR)], sc_v)
            copies = []
            for i in range(_CH):
                copies.append(pltpu.async_copy(
                    table_hbm.at[nid_v.at[i]], rows_v.at[i], gsem))
            self_cp = pltpu.async_copy(
                table_hbm.at[eid_v.at[pl.ds(c * _CH, _CH)]], self_v, csem)
            for i in range(_CH):
                evs = []
                mx = None
                for j in range(NS // _L):
                    r = rid_v[i, pl.ds(j * _L, _L)] + jnp.int32(i * NR)
                    sg = plsc.load_gather(sc_v, [r])
                    evs.append(sg)
                    m = jnp.max(sg)
                    mx = m if mx is None else jnp.maximum(mx, m)
                den = jnp.float32(0.0)
                for j in range(NS // _L):
                    evs[j] = jnp.exp(evs[j] - mx)
                    den = den + jnp.sum(evs[j])
                denv = jnp.full((_L,), 1.0, jnp.float32) * den
                for j in range(NS // _L):
                    wbuf[pl.ds(i * NS + j * _L, _L)] = evs[j] / denv
            for cp in copies:
                cp.wait()
            self_cp.wait()
            for i in range(_CH):
                def nbody(s, acc):
                    w = plsc.load_gather(
                        wbuf, [jnp.full((_L,), i * NS, dtype=jnp.int32) + s])
                    return tuple(
                        acc[d] + w * rows_v[i, s, pl.ds(d * _L, _L)]
                        for d in range(DV))
                acc = lax.fori_loop(
                    0, NS, nbody,
                    tuple(jnp.zeros((_L,), jnp.float32) for _ in range(DV)))
                for d in range(DV):
                    out_v[i, pl.ds(d * _L, _L)] = jnp.maximum(
                        acc[d] + self_v[i, pl.ds(d * _L, _L)], 0.0)
            pltpu.sync_copy(out_v, out_hbm.at[pl.ds(base, _CH)])
            return ()

        lax.fori_loop(0, n_chunks, chunk_body, ())

    return k(all_scores.reshape(B * NR), entity_ids, neigh_ent_ids,
             neigh_rel_ids, entity_table)


def kernel(user_emb, entity_ids, neigh_ent_ids, neigh_rel_ids, entity_table,
           relation_table, W):
    all_scores = _all_scores(user_emb, W, relation_table)
    return _sc_agg(
        all_scores,
        entity_ids.astype(jnp.int32),
        neigh_ent_ids.astype(jnp.int32),
        neigh_rel_ids.astype(jnp.int32),
        entity_table,
    )
